# baseline (speedup 1.0000x reference)
import functools

import jax
import jax.numpy as jnp
from jax import lax
from jax.experimental import pallas as pl
from jax.experimental.pallas import tpu as pltpu
from jax.experimental.pallas import tpu_sc as plsc

N = 10000
E = 160000
D = 256

K = 128
EH = E // 2
ECH = EH // K
ECPT = (ECH + 15) // 16
NZF = N // K
NZR = N - NZF * K
NZPT = (NZF + 16) // 16

MBLK = 400
NB = N // MBLK


def _mesh():
    return plsc.VectorSubcoreMesh(core_axis_name="c", subcore_axis_name="s")


NCH = E // K
DCPT = (NCH + 31) // 32
NBANK = 8


def _sc_deg(edge_index):
    @functools.partial(
        pl.kernel,
        mesh=_mesh(),
        out_type=jax.ShapeDtypeStruct((64, N), jnp.float32),
        compiler_params=pltpu.CompilerParams(needs_layout_passes=False,
                                             use_tc_tiling_on_sc=False),
        scratch_types=[
            pltpu.VMEM((E // 32 + 16,), jnp.int32),
            pltpu.VMEM((NBANK * N + 16,), jnp.float32),
        ],
    )
    def k(ei_hbm, out_hbm, idxv, hist):
        EPW = E // 32
        c = lax.axis_index("c")
        s = lax.axis_index("s")
        w = s * 2 + c
        iota = lax.iota(jnp.int32, 16)
        bank = (iota & 7) * N
        ones16 = jnp.ones((16,), jnp.float32)
        zero16 = jnp.zeros((16,), jnp.float32)
        mlo = iota < 8
        mhi = iota >= 8
        dump = NBANK * N + iota

        def histo(row_sel, out_row):
            def zb(j, _):
                for g in range(8):
                    hist[pl.ds(j * 128 + g * 16, 16)] = zero16
                return 0

            lax.fori_loop(0, (NBANK * N) // 128, zb, 0)
            hist[pl.ds(NBANK * N, 16)] = zero16

            pltpu.sync_copy(ei_hbm.at[row_sel, pl.ds(w * EPW, EPW)],
                            idxv.at[pl.ds(0, EPW)])

            def grp(j, _):
                for u in range(2):
                    idx = bank + idxv[pl.ds(j * 32 + u * 16, 16)]
                    plsc.addupdate_scatter(
                        hist, [jnp.where(mlo, idx, dump)], ones16)
                    plsc.addupdate_scatter(
                        hist, [jnp.where(mhi, idx, dump)], ones16)
                return 0

            lax.fori_loop(0, EPW // 32, grp, 0)
            vt = idxv[pl.ds(EPW - EPW % 32, 16)]
            plsc.addupdate_scatter(
                hist, [jnp.where(mlo, bank + vt, dump)], ones16)

            def red(j, _):
                sl = pl.ds(j * 16, 16)
                acc = hist[sl]
                for bk in range(1, NBANK):
                    acc = acc + hist[pl.ds(bk * N + j * 16, 16)]
                hist[sl] = acc
                return 0

            lax.fori_loop(0, N // 16, red, 0)
            pltpu.sync_copy(hist.at[pl.ds(0, N)], out_hbm.at[out_row])

        histo(0, w)
        histo(1, 32 + w)

    return k(edge_index)


DH = D // 2
ACCROWS = N + 8
KA = 128
EPAD = 163840
EPT = EPAD // 16
SEG = 2048
NSEG = EPT // SEG
CPS = SEG // KA
ZF = N // KA


def _sc_agg(h2, ei_pad):
    @functools.partial(
        pl.kernel,
        mesh=_mesh(),
        out_type=(jax.ShapeDtypeStruct((N, DH), jnp.float32),
                  jax.ShapeDtypeStruct((N, DH), jnp.float32)),
        compiler_params=pltpu.CompilerParams(needs_layout_passes=False,
                                             use_tc_tiling_on_sc=False),
        scratch_types=[
            pltpu.VMEM((SEG,), jnp.int32),
            pltpu.VMEM((SEG,), jnp.int32),
            pltpu.VMEM((KA,), jnp.int32),
            pltpu.VMEM((KA,), jnp.int32),
            pltpu.VMEM((KA,), jnp.int32),
            pltpu.VMEM((KA,), jnp.int32),
            pltpu.VMEM((KA, DH), jnp.float32),
            pltpu.VMEM((KA, DH), jnp.float32),
            pltpu.VMEM_SHARED((ACCROWS, DH), jnp.float32),
            pltpu.SemaphoreType.DMA,
            pltpu.SemaphoreType.DMA,
            pltpu.SemaphoreType.DMA,
            pltpu.SemaphoreType.DMA,
        ],
    )
    def k(h_hbm, ei_hbm, out0_hbm, out1_hbm, sidx_seg, didx_seg,
          gidx0, gidx1, didx0, didx1, rows0, rows1, acc,
          gsem0, gsem1, ssem0, ssem1):
        c = lax.axis_index("c")
        s = lax.axis_index("s")
        zero16 = jnp.zeros((16,), jnp.float32)
        rows = (rows0, rows1)
        gidx = (gidx0, gidx1)
        didx = (didx0, didx1)
        gsem = (gsem0, gsem1)
        ssem = (ssem0, ssem1)
        t0 = s * EPT

        def zrow(r, _):
            for cc in range(0, DH, 16):
                rows0[r, pl.ds(cc, 16)] = zero16
            return 0

        lax.fori_loop(0, KA, zrow, 0)

        def zchunk(i, _):
            ch = i * 16 + s

            @pl.when(ch < ZF)
            def _():
                pltpu.sync_copy(rows0, acc.at[pl.ds(ch * KA, KA)])

            @pl.when(ch == ZF)
            def _():
                pltpu.sync_copy(rows0.at[pl.ds(0, ACCROWS - ZF * KA)],
                                acc.at[pl.ds(ZF * KA, ACCROWS - ZF * KA)])

            return 0

        lax.fori_loop(0, (ZF + 16) // 16, zchunk, 0)
        plsc.subcore_barrier()

        def gather_start(j):
            pltpu.make_async_copy(h_hbm.at[gidx[j]], rows[j], gsem[j]
                                  ).start()

        def gather_wait(j):
            pltpu.make_async_copy(h_hbm.at[gidx[j]], rows[j], gsem[j]
                                  ).wait()

        def scatter_start(j):
            pltpu.make_async_copy(rows[j], acc.at[didx[j]], ssem[j]
                                  ).start(add=True)

        def scatter_wait(j):
            pltpu.make_async_copy(rows[j], acc.at[didx[j]], ssem[j]).wait()

        def load_ids(p, j):
            for g in range(KA // 16):
                sl = pl.ds(g * 16, 16)
                sv = sidx_seg[pl.ds(p * KA + g * 16, 16)]
                gidx[j][sl] = sv * 2 + c
                didx[j][sl] = didx_seg[pl.ds(p * KA + g * 16, 16)]

        def phase(p, j):
            @pl.when(p >= 1)
            def _():
                scatter_wait(1 - j)

            @pl.when(p + 1 < CPS)
            def _():
                load_ids(p + 1, 1 - j)
                gather_start(1 - j)

            gather_wait(j)
            scatter_start(j)

        for seg in range(NSEG):
            e0 = t0 + seg * SEG
            pltpu.sync_copy(ei_hbm.at[0, pl.ds(e0, SEG)], sidx_seg)
            pltpu.sync_copy(ei_hbm.at[1, pl.ds(e0, SEG)], didx_seg)
            if seg > 0:
                scatter_wait(1)
            load_ids(0, 0)
            gather_start(0)

            def rounds(r, _):
                phase(2 * r, 0)
                phase(2 * r + 1, 1)
                return 0

            lax.fori_loop(0, CPS // 2, rounds, 0)

        scatter_wait(1)
        plsc.subcore_barrier()

        def wb(src_off, n):
            pltpu.sync_copy(acc.at[pl.ds(src_off, n)],
                            rows0.at[pl.ds(0, n)])

            @pl.when(c == 0)
            def _():
                pltpu.sync_copy(rows0.at[pl.ds(0, n)],
                                out0_hbm.at[pl.ds(src_off, n)])

            @pl.when(c == 1)
            def _():
                pltpu.sync_copy(rows0.at[pl.ds(0, n)],
                                out1_hbm.at[pl.ds(src_off, n)])

        def wchunk(i, _):
            ch = i * 16 + s

            @pl.when(ch < ZF)
            def _():
                wb(ch * KA, KA)

            @pl.when(ch == ZF)
            def _():
                wb(ZF * KA, N - ZF * KA)

            return 0

        lax.fori_loop(0, (ZF + 16) // 16, wchunk, 0)

    return k(h2, ei_pad)


NP128 = 10240


def _tc_norms(degs_p):
    def body(d_ref, ns_ref, nd_ref):
        d = d_ref[...]
        ones = jnp.ones((32, 128), jnp.float32)

        def col(rows):
            deg = lax.dot_general(
                rows, ones, (((0,), (0,)), ((), ())),
                preferred_element_type=jnp.float32,
                precision=lax.Precision.HIGHEST,
            )[:, 0:1]
            return lax.rsqrt(jnp.maximum(deg, 1.0))

        ns_ref[...] = col(d[0:32])
        nd_ref[...] = col(d[32:64])

    out = pl.pallas_call(
        body,
        grid=(NP128 // 128,),
        in_specs=[pl.BlockSpec((64, 128), lambda i: (0, i))],
        out_specs=[pl.BlockSpec((128, 1), lambda i: (i, 0)),
                   pl.BlockSpec((128, 1), lambda i: (i, 0))],
        out_shape=[jax.ShapeDtypeStruct((NP128, 1), jnp.float32),
                   jax.ShapeDtypeStruct((NP128, 1), jnp.float32)],
    )(degs_p)
    return out


def _tc_scale(h, ns):
    def body(h_ref, n_ref, o_ref):
        o_ref[...] = h_ref[...] * n_ref[...]

    return pl.pallas_call(
        body,
        grid=(NB,),
        in_specs=[
            pl.BlockSpec((MBLK, D), lambda i: (i, 0)),
            pl.BlockSpec((MBLK, 1), lambda i: (i, 0)),
        ],
        out_specs=pl.BlockSpec((MBLK, D), lambda i: (i, 0)),
        out_shape=jax.ShapeDtypeStruct((N, D), jnp.float32),
    )(h, ns)


def _tc_mm(p0, p1, W, b, nd, ns):
    scale_out = ns is not None

    def body(pa_ref, pb_ref, w_ref, b_ref, nd_ref, *rest):
        if scale_out:
            ns_ref, o_ref = rest
        else:
            (o_ref,) = rest
        y = jnp.dot(pa_ref[...], w_ref[0:DH, :],
                    preferred_element_type=jnp.float32)
        y = y + jnp.dot(pb_ref[...], w_ref[DH:D, :],
                        preferred_element_type=jnp.float32)
        y = y * nd_ref[...] + b_ref[...]
        if scale_out:
            y = y * ns_ref[...]
        o_ref[...] = y

    in_specs = [
        pl.BlockSpec((MBLK, DH), lambda i: (i, 0)),
        pl.BlockSpec((MBLK, DH), lambda i: (i, 0)),
        pl.BlockSpec((D, D), lambda i: (0, 0)),
        pl.BlockSpec((1, D), lambda i: (0, 0)),
        pl.BlockSpec((MBLK, 1), lambda i: (i, 0)),
    ]
    args = [p0, p1, W, b[None, :], nd]
    if scale_out:
        in_specs.append(pl.BlockSpec((MBLK, 1), lambda i: (i, 0)))
        args.append(ns)
    return pl.pallas_call(
        body,
        grid=(NB,),
        in_specs=in_specs,
        out_specs=pl.BlockSpec((MBLK, D), lambda i: (i, 0)),
        out_shape=jax.ShapeDtypeStruct((N, D), jnp.float32),
    )(*args)


def _sc_scatter_dbg(msg, edge_index):
    @functools.partial(
        pl.kernel,
        mesh=_mesh(),
        out_type=jax.ShapeDtypeStruct((2 * N, D), jnp.float32),
        scratch_types=[
            pltpu.VMEM((K,), jnp.int32),
            pltpu.VMEM((K, D), jnp.float32),
        ],
    )
    def k(msg_hbm, ei_hbm, out_hbm, didx, rows):
        c = lax.axis_index("c")
        s = lax.axis_index("s")
        zero16 = jnp.zeros((16,), jnp.float32)

        def zrow(r, _):
            for cc in range(0, D, 16):
                rows[r, pl.ds(cc, 16)] = zero16
            return 0

        lax.fori_loop(0, K, zrow, 0)

        def zchunk(i, _):
            ch = i * 16 + s

            @pl.when(ch < NZF)
            def _():
                pltpu.sync_copy(rows, out_hbm.at[pl.ds(c * N + ch * K, K)])

            @pl.when(ch == NZF)
            def _():
                pltpu.sync_copy(rows.at[pl.ds(0, NZR)],
                                out_hbm.at[pl.ds(c * N + NZF * K, NZR)])

            return 0

        lax.fori_loop(0, NZPT, zchunk, 0)
        plsc.subcore_barrier()

        def chunk(i, _):
            ch = i

            @pl.when((ch < ECH) & (s == 0))
            def _():
                e0 = c * EH + ch * K
                pltpu.sync_copy(msg_hbm.at[pl.ds(e0, K)], rows)
                pltpu.sync_copy(ei_hbm.at[1, pl.ds(e0, K)], didx)
                for g in range(K // 16):
                    sl = pl.ds(g * 16, 16)
                    didx[sl] = didx[sl] + c * N
                pltpu.sync_copy(rows, out_hbm.at[didx], add=True)

            return 0

        lax.fori_loop(0, ECH, chunk, 0)

    return k(msg, edge_index)


def _jnp_agg(h, edge_index):
    msg = jnp.take(h, edge_index[0], axis=0)
    agg = jax.ops.segment_sum(msg, edge_index[1], num_segments=N)
    return jnp.concatenate([agg, jnp.zeros((N, D), jnp.float32)], axis=0)


def kernel(in_feat, edge_index, W, b):
    npad = EPAD - E
    pad = jnp.stack([jnp.zeros((npad,), jnp.int32),
                     jnp.full((npad,), N, jnp.int32)])
    ei_pad = jnp.concatenate([edge_index, pad], axis=1)

    degs = _sc_deg(edge_index)
    degs_p = jnp.pad(degs, ((0, 0), (0, NP128 - N)))
    ns, nd = _tc_norms(degs_p)
    h1 = _tc_scale(in_feat, ns)
    p1a, p1b = _sc_agg(h1.reshape(2 * N, DH), ei_pad)
    h2 = _tc_mm(p1a, p1b, W, b, nd, ns)
    p2a, p2b = _sc_agg(h2.reshape(2 * N, DH), ei_pad)
    return _tc_mm(p2a, p2b, W, b, nd, None)


# speedup vs baseline: 1.3035x; 1.3035x over previous
import functools

import jax
import jax.numpy as jnp
from jax import lax
from jax.experimental import pallas as pl
from jax.experimental.pallas import tpu as pltpu
from jax.experimental.pallas import tpu_sc as plsc

N = 10000
E = 160000
D = 256

K = 128
EH = E // 2
ECH = EH // K
ECPT = (ECH + 15) // 16
NZF = N // K
NZR = N - NZF * K
NZPT = (NZF + 16) // 16

MBLK = 400
NB = N // MBLK


def _mesh():
    return plsc.VectorSubcoreMesh(core_axis_name="c", subcore_axis_name="s")


NCH = E // K
DCPT = (NCH + 31) // 32
NBANK = 8


def _sc_deg(edge_index):
    @functools.partial(
        pl.kernel,
        mesh=_mesh(),
        out_type=jax.ShapeDtypeStruct((64, N), jnp.float32),
        compiler_params=pltpu.CompilerParams(needs_layout_passes=False,
                                             use_tc_tiling_on_sc=False),
        scratch_types=[
            pltpu.VMEM((E // 32 + 16,), jnp.int32),
            pltpu.VMEM((NBANK * N + 16,), jnp.float32),
        ],
    )
    def k(ei_hbm, out_hbm, idxv, hist):
        EPW = E // 32
        c = lax.axis_index("c")
        s = lax.axis_index("s")
        w = s * 2 + c
        iota = lax.iota(jnp.int32, 16)
        bank = (iota & 7) * N
        ones16 = jnp.ones((16,), jnp.float32)
        zero16 = jnp.zeros((16,), jnp.float32)
        mlo = iota < 8
        mhi = iota >= 8
        dump = NBANK * N + iota

        def histo(row_sel, out_row):
            def zb(j, _):
                for g in range(8):
                    hist[pl.ds(j * 128 + g * 16, 16)] = zero16
                return 0

            lax.fori_loop(0, (NBANK * N) // 128, zb, 0)
            hist[pl.ds(NBANK * N, 16)] = zero16

            pltpu.sync_copy(ei_hbm.at[row_sel, pl.ds(w * EPW, EPW)],
                            idxv.at[pl.ds(0, EPW)])

            def grp(j, _):
                for u in range(2):
                    idx = bank + idxv[pl.ds(j * 32 + u * 16, 16)]
                    plsc.addupdate_scatter(
                        hist, [jnp.where(mlo, idx, dump)], ones16)
                    plsc.addupdate_scatter(
                        hist, [jnp.where(mhi, idx, dump)], ones16)
                return 0

            lax.fori_loop(0, EPW // 32, grp, 0)
            vt = idxv[pl.ds(EPW - EPW % 32, 16)]
            plsc.addupdate_scatter(
                hist, [jnp.where(mlo, bank + vt, dump)], ones16)

            def red(j, _):
                sl = pl.ds(j * 16, 16)
                acc = hist[sl]
                for bk in range(1, NBANK):
                    acc = acc + hist[pl.ds(bk * N + j * 16, 16)]
                hist[sl] = acc
                return 0

            lax.fori_loop(0, N // 16, red, 0)
            pltpu.sync_copy(hist.at[pl.ds(0, N)], out_hbm.at[out_row])

        histo(0, w)
        histo(1, 32 + w)

    return k(edge_index)


HALF = N // 2
DUMP = HALF
ACCROWS = HALF + 8
KA = 80
EPT = E // 16
SEG = 2000
NSEG = EPT // SEG
CPS = SEG // KA
ZF = HALF // KA


def _sc_agg(h, edge_index):
    @functools.partial(
        pl.kernel,
        mesh=_mesh(),
        out_type=jax.ShapeDtypeStruct((N, D), jnp.float32),
        compiler_params=pltpu.CompilerParams(needs_layout_passes=False,
                                             use_tc_tiling_on_sc=False),
        scratch_types=[
            pltpu.VMEM((SEG,), jnp.int32),
            pltpu.VMEM((SEG,), jnp.int32),
            pltpu.VMEM((KA,), jnp.int32),
            pltpu.VMEM((KA,), jnp.int32),
            pltpu.VMEM((KA, D), jnp.float32),
            pltpu.VMEM((KA, D), jnp.float32),
            pltpu.VMEM_SHARED((ACCROWS, D), jnp.float32),
            pltpu.SemaphoreType.DMA,
            pltpu.SemaphoreType.DMA,
            pltpu.SemaphoreType.DMA,
            pltpu.SemaphoreType.DMA,
        ],
    )
    def k(h_hbm, ei_hbm, out_hbm, sidx_seg, didx_seg,
          didx0, didx1, rows0, rows1, acc, gsem0, gsem1, ssem0, ssem1):
        c = lax.axis_index("c")
        s = lax.axis_index("s")
        lo = c * HALF
        zero16 = jnp.zeros((16,), jnp.float32)
        rows = (rows0, rows1)
        didx = (didx0, didx1)
        gsem = (gsem0, gsem1)
        ssem = (ssem0, ssem1)
        t0 = s * EPT

        def zrow(r, _):
            for cc in range(0, D, 16):
                rows0[r, pl.ds(cc, 16)] = zero16
            return 0

        lax.fori_loop(0, KA, zrow, 0)

        def zchunk(i, _):
            ch = i * 16 + s

            @pl.when(ch < ZF)
            def _():
                pltpu.sync_copy(rows0, acc.at[pl.ds(ch * KA, KA)])

            @pl.when(ch == ZF)
            def _():
                pltpu.sync_copy(rows0.at[pl.ds(0, ACCROWS - ZF * KA)],
                                acc.at[pl.ds(ZF * KA, ACCROWS - ZF * KA)])

            return 0

        lax.fori_loop(0, (ZF + 16) // 16, zchunk, 0)
        plsc.subcore_barrier()

        def gather_start(p, j):
            pltpu.make_async_copy(
                h_hbm.at[sidx_seg.at[pl.ds(p * KA, KA)]], rows[j], gsem[j]
            ).start()

        def gather_wait(p, j):
            pltpu.make_async_copy(
                h_hbm.at[sidx_seg.at[pl.ds(p * KA, KA)]], rows[j], gsem[j]
            ).wait()

        def scatter_start(j):
            pltpu.make_async_copy(rows[j], acc.at[didx[j]], ssem[j]
                                  ).start(add=True)

        def scatter_wait(j):
            pltpu.make_async_copy(rows[j], acc.at[didx[j]], ssem[j]).wait()

        def phase(p, j, seg):
            @pl.when(p >= 1)
            def _():
                scatter_wait(1 - j)

            @pl.when(p + 1 < CPS)
            def _():
                gather_start(p + 1, 1 - j)

            gather_wait(p, j)
            for g in range(KA // 16):
                d = didx_seg[pl.ds(p * KA + g * 16, 16)] - lo
                ok = (d >= 0) & (d < HALF)
                didx[j][pl.ds(g * 16, 16)] = jnp.where(ok, d, DUMP)
            scatter_start(j)

        for seg in range(NSEG):
            e0 = t0 + seg * SEG
            pltpu.sync_copy(ei_hbm.at[0, pl.ds(e0, SEG)], sidx_seg)
            pltpu.sync_copy(ei_hbm.at[1, pl.ds(e0, SEG)], didx_seg)
            if seg > 0:
                scatter_wait(0)
            gather_start(0, 0)

            def rounds(r, _):
                phase(2 * r, 0, seg)

                @pl.when(2 * r + 1 < CPS)
                def _():
                    phase(2 * r + 1, 1, seg)

                return 0

            lax.fori_loop(0, (CPS + 1) // 2, rounds, 0)

        scatter_wait(0)
        plsc.subcore_barrier()

        def wchunk(i, _):
            ch = i * 16 + s

            @pl.when(ch < ZF)
            def _():
                pltpu.sync_copy(acc.at[pl.ds(ch * KA, KA)], rows0)
                pltpu.sync_copy(rows0, out_hbm.at[pl.ds(lo + ch * KA, KA)])

            @pl.when(ch == ZF)
            def _():
                nrem = HALF - ZF * KA
                pltpu.sync_copy(acc.at[pl.ds(ZF * KA, nrem)],
                                rows0.at[pl.ds(0, nrem)])
                pltpu.sync_copy(rows0.at[pl.ds(0, nrem)],
                                out_hbm.at[pl.ds(lo + ZF * KA, nrem)])

            return 0

        lax.fori_loop(0, (ZF + 16) // 16, wchunk, 0)

    return k(h, edge_index)


NP128 = 10240


def _tc_norms(degs_p):
    def body(d_ref, ns_ref, nd_ref):
        d = d_ref[...]
        ones = jnp.ones((32, 128), jnp.float32)

        def col(rows):
            deg = lax.dot_general(
                rows, ones, (((0,), (0,)), ((), ())),
                preferred_element_type=jnp.float32,
                precision=lax.Precision.HIGHEST,
            )[:, 0:1]
            return lax.rsqrt(jnp.maximum(deg, 1.0))

        ns_ref[...] = col(d[0:32])
        nd_ref[...] = col(d[32:64])

    out = pl.pallas_call(
        body,
        grid=(NP128 // 128,),
        in_specs=[pl.BlockSpec((64, 128), lambda i: (0, i))],
        out_specs=[pl.BlockSpec((128, 1), lambda i: (i, 0)),
                   pl.BlockSpec((128, 1), lambda i: (i, 0))],
        out_shape=[jax.ShapeDtypeStruct((NP128, 1), jnp.float32),
                   jax.ShapeDtypeStruct((NP128, 1), jnp.float32)],
    )(degs_p)
    return out


def _tc_scale(h, ns):
    def body(h_ref, n_ref, o_ref):
        o_ref[...] = h_ref[...] * n_ref[...]

    return pl.pallas_call(
        body,
        grid=(NB,),
        in_specs=[
            pl.BlockSpec((MBLK, D), lambda i: (i, 0)),
            pl.BlockSpec((MBLK, 1), lambda i: (i, 0)),
        ],
        out_specs=pl.BlockSpec((MBLK, D), lambda i: (i, 0)),
        out_shape=jax.ShapeDtypeStruct((N, D), jnp.float32),
    )(h, ns)


def _tc_mm(p, W, b, nd, ns):
    scale_out = ns is not None

    def body(pa_ref, w_ref, b_ref, nd_ref, *rest):
        if scale_out:
            ns_ref, o_ref = rest
        else:
            (o_ref,) = rest
        y = jnp.dot(pa_ref[...], w_ref[...],
                    preferred_element_type=jnp.float32)
        y = y * nd_ref[...] + b_ref[...]
        if scale_out:
            y = y * ns_ref[...]
        o_ref[...] = y

    in_specs = [
        pl.BlockSpec((MBLK, D), lambda i: (i, 0)),
        pl.BlockSpec((D, D), lambda i: (0, 0)),
        pl.BlockSpec((1, D), lambda i: (0, 0)),
        pl.BlockSpec((MBLK, 1), lambda i: (i, 0)),
    ]
    args = [p, W, b[None, :], nd]
    if scale_out:
        in_specs.append(pl.BlockSpec((MBLK, 1), lambda i: (i, 0)))
        args.append(ns)
    return pl.pallas_call(
        body,
        grid=(NB,),
        in_specs=in_specs,
        out_specs=pl.BlockSpec((MBLK, D), lambda i: (i, 0)),
        out_shape=jax.ShapeDtypeStruct((N, D), jnp.float32),
    )(*args)


def _sc_scatter_dbg(msg, edge_index):
    @functools.partial(
        pl.kernel,
        mesh=_mesh(),
        out_type=jax.ShapeDtypeStruct((2 * N, D), jnp.float32),
        scratch_types=[
            pltpu.VMEM((K,), jnp.int32),
            pltpu.VMEM((K, D), jnp.float32),
        ],
    )
    def k(msg_hbm, ei_hbm, out_hbm, didx, rows):
        c = lax.axis_index("c")
        s = lax.axis_index("s")
        zero16 = jnp.zeros((16,), jnp.float32)

        def zrow(r, _):
            for cc in range(0, D, 16):
                rows[r, pl.ds(cc, 16)] = zero16
            return 0

        lax.fori_loop(0, K, zrow, 0)

        def zchunk(i, _):
            ch = i * 16 + s

            @pl.when(ch < NZF)
            def _():
                pltpu.sync_copy(rows, out_hbm.at[pl.ds(c * N + ch * K, K)])

            @pl.when(ch == NZF)
            def _():
                pltpu.sync_copy(rows.at[pl.ds(0, NZR)],
                                out_hbm.at[pl.ds(c * N + NZF * K, NZR)])

            return 0

        lax.fori_loop(0, NZPT, zchunk, 0)
        plsc.subcore_barrier()

        def chunk(i, _):
            ch = i

            @pl.when((ch < ECH) & (s == 0))
            def _():
                e0 = c * EH + ch * K
                pltpu.sync_copy(msg_hbm.at[pl.ds(e0, K)], rows)
                pltpu.sync_copy(ei_hbm.at[1, pl.ds(e0, K)], didx)
                for g in range(K // 16):
                    sl = pl.ds(g * 16, 16)
                    didx[sl] = didx[sl] + c * N
                pltpu.sync_copy(rows, out_hbm.at[didx], add=True)

            return 0

        lax.fori_loop(0, ECH, chunk, 0)

    return k(msg, edge_index)


def _jnp_agg(h, edge_index):
    msg = jnp.take(h, edge_index[0], axis=0)
    agg = jax.ops.segment_sum(msg, edge_index[1], num_segments=N)
    return jnp.concatenate([agg, jnp.zeros((N, D), jnp.float32)], axis=0)


def kernel(in_feat, edge_index, W, b):
    degs = _sc_deg(edge_index)
    degs_p = jnp.pad(degs, ((0, 0), (0, NP128 - N)))
    ns, nd = _tc_norms(degs_p)
    h1 = _tc_scale(in_feat, ns)
    p1 = _sc_agg(h1, edge_index)
    h2 = _tc_mm(p1, W, b, nd, ns)
    p2 = _sc_agg(h2, edge_index)
    return _tc_mm(p2, W, b, nd, None)
